# baseline (speedup 1.0000x reference)
import jax
import jax.numpy as jnp
from jax import lax
from jax.experimental import pallas as pl
from jax.experimental.pallas import tpu as pltpu

N = 256
A = 8
F = 8
FE = 16
SP = 32
P = 32
AAF = A * A * F
BP = 16

_F32 = jnp.float32


def _fused_kernel(x_hbm, f1_ref, geoT_ref, z_ref, k2_ref, w1b_ref,
                  slot_row_ref, slot_col_ref,
                  atomf_ref, efeat_ref, edist_ref, eunit_ref, nidx_ref, nz_ref,
                  xbuf, xsem):
    pid = pl.program_id(0)
    nsteps = pl.num_programs(0)
    cur = lax.rem(pid, 2)
    nxt = lax.rem(pid + 1, 2)

    @pl.when(pid == 0)
    def _():
        pltpu.make_async_copy(x_hbm.at[pl.ds(0, BP)], xbuf.at[0],
                              xsem.at[0]).start()

    @pl.when(pid + 1 < nsteps)
    def _():
        pltpu.make_async_copy(x_hbm.at[pl.ds((pid + 1) * BP, BP)],
                              xbuf.at[nxt], xsem.at[nxt]).start()

    pltpu.make_async_copy(x_hbm.at[pl.ds(pid * BP, BP)], xbuf.at[cur],
                          xsem.at[cur]).wait()

    atomf_ref[...] = jnp.tanh(
        jnp.dot(f1_ref[...], w1b_ref[...], preferred_element_type=_F32))

    x = xbuf[cur].reshape(BP * N, AAF)
    folded = jnp.dot(x, k2_ref[...],
                     preferred_element_type=_F32)

    geoT = geoT_ref[...]
    zf = z_ref[...]
    slot_row = slot_row_ref[...]
    slot_col = slot_col_ref[...]

    col_i32 = lax.broadcasted_iota(jnp.int32, (1, N), 1)
    colf = col_i32.astype(_F32)
    s_colf = lax.broadcasted_iota(jnp.int32, (P, 1), 0).astype(_F32)
    s_rowf = lax.broadcasted_iota(jnp.int32, (1, P), 1).astype(_F32)
    fselT = ((lax.broadcasted_iota(jnp.int32, (F, AAF), 1) & 7)
             == lax.broadcasted_iota(jnp.int32, (F, AAF), 0)).astype(_F32)
    tri = (lax.broadcasted_iota(jnp.int32, (N, N), 0)
           <= lax.broadcasted_iota(jnp.int32, (N, N), 1)).astype(_F32)

    vm_rows = []
    for i in range(BP):
        fi = folded[i * N:(i + 1) * N, :]
        sq = fi * fi
        n2T = lax.dot_general(fselT, sq, (((1,), (1,)), ((), ())),
                              precision=lax.Precision.DEFAULT,
                              preferred_element_type=_F32)
        mx = jnp.max(n2T, axis=0, keepdims=True)
        ao = -jnp.log(jnp.sqrt(mx) + 1e-6)
        vm = jnp.logical_and(ao < 12.0, col_i32 != pid * BP + i)
        vm_rows.append(vm.astype(_F32))
    m = jnp.concatenate(vm_rows, axis=0)

    vmask = m > 0.5
    cs = jnp.dot(m, tri, preferred_element_type=_F32)
    t = cs[:, N - 1:N]
    rank = jnp.where(vmask, cs - 1.0, t + colf - cs)

    for i in range(BP):
        rank_i = rank[i:i + 1, :]
        oh = (rank_i == s_colf).astype(_F32)
        t_i = t[i:i + 1, :]
        padc = jnp.where(s_colf < t_i, 1.0, 0.0) * slot_col
        padr = jnp.where(s_rowf < t_i, 1.0, 0.0) * slot_row

        fi = folded[i * N:(i + 1) * N, :]
        efeat_ref[i] = jnp.dot(oh, fi, precision=lax.Precision.DEFAULT,
                               preferred_element_type=_F32) * padc

        g_i = jnp.sum(jnp.where(col_i32 == pid * BP + i, geoT, 0.0),
                      axis=1, keepdims=True)
        diffT = geoT - g_i
        dist = jnp.sqrt(jnp.sum(diffT * diffT, axis=0, keepdims=True)
                        + 1e-12)
        vm_i = vmask[i:i + 1, :]
        denom = jnp.where(vm_i, dist, 1.0)
        unitT = jnp.where(vm_i, diffT / denom, 0.0)

        eunit_ref[i] = lax.dot_general(
            oh, unitT, (((1,), (1,)), ((), ())),
            preferred_element_type=_F32) * padc
        edist_ref[pl.ds(i, 1), :] = lax.dot_general(
            dist, oh, (((1,), (1,)), ((), ())),
            preferred_element_type=_F32) * padr
        nidx = lax.dot_general(colf, oh, (((1,), (1,)), ((), ())),
                               preferred_element_type=_F32)
        nidx_ref[pl.ds(i, 1), :] = jnp.round(nidx).astype(jnp.int32)
        nz = lax.dot_general(zf, oh, (((1,), (1,)), ((), ())),
                             preferred_element_type=_F32) * padr
        nz_ref[pl.ds(i, 1), :] = jnp.round(nz).astype(jnp.int32)


def kernel(feat_2body, feat_1body, geometry, atomic_numbers,
           W_fold1, W_fold2, W_onebody, padding_size):
    n = feat_2body.shape[0]
    x2b = feat_2body.reshape(n, n, AAF).astype(jnp.bfloat16)
    geoT = geometry.T.astype(_F32)
    zf = atomic_numbers.astype(_F32).reshape(1, n)
    k2 = jnp.kron(jnp.kron(W_fold1, W_fold2),
                  jnp.eye(F, dtype=_F32)).astype(jnp.bfloat16)
    slot_row = (jnp.arange(P)[None, :] < padding_size).astype(_F32)
    slot_col = slot_row.reshape(P, 1)

    grid = (n // BP,)
    outs = pl.pallas_call(
        _fused_kernel,
        grid=grid,
        in_specs=[
            pl.BlockSpec(memory_space=pltpu.MemorySpace.HBM),
            pl.BlockSpec((BP, FE), lambda i: (i, 0)),
            pl.BlockSpec((3, n), lambda i: (0, 0)),
            pl.BlockSpec((1, n), lambda i: (0, 0)),
            pl.BlockSpec((AAF, AAF), lambda i: (0, 0)),
            pl.BlockSpec((FE, SP), lambda i: (0, 0)),
            pl.BlockSpec((1, P), lambda i: (0, 0)),
            pl.BlockSpec((P, 1), lambda i: (0, 0)),
        ],
        out_specs=[
            pl.BlockSpec((BP, SP), lambda i: (i, 0)),
            pl.BlockSpec((BP, P, AAF), lambda i: (i, 0, 0)),
            pl.BlockSpec((BP, P), lambda i: (i, 0)),
            pl.BlockSpec((BP, P, 3), lambda i: (i, 0, 0)),
            pl.BlockSpec((BP, P), lambda i: (i, 0)),
            pl.BlockSpec((BP, P), lambda i: (i, 0)),
        ],
        out_shape=[
            jax.ShapeDtypeStruct((n, SP), _F32),
            jax.ShapeDtypeStruct((n, P, AAF), _F32),
            jax.ShapeDtypeStruct((n, P), _F32),
            jax.ShapeDtypeStruct((n, P, 3), _F32),
            jax.ShapeDtypeStruct((n, P), jnp.int32),
            jax.ShapeDtypeStruct((n, P), jnp.int32),
        ],
        scratch_shapes=[
            pltpu.VMEM((2, BP, N, AAF), jnp.bfloat16),
            pltpu.SemaphoreType.DMA((2,)),
        ],
        compiler_params=pltpu.CompilerParams(
            dimension_semantics=("arbitrary",)),
    )(x2b, feat_1body, geoT, zf, k2, W_onebody, slot_row, slot_col)

    atom_f, ef, edist, eunit, nidx, nz = outs
    return (atom_f, ef.reshape(n, P, A, A, F), edist, eunit, nidx, nz)


# speedup vs baseline: 1.4424x; 1.4424x over previous
import jax
import jax.numpy as jnp
from jax import lax
from jax.experimental import pallas as pl
from jax.experimental.pallas import tpu as pltpu

N = 256
A = 8
F = 8
FE = 16
SP = 32
P = 32
AAF = A * A * F
BP = 16

_F32 = jnp.float32


def _fused_kernel(x_ref, f1_ref, geoT_ref, z_ref, w1_ref, w2_ref, w1b_ref,
                  slot_row_ref, slot_col_ref,
                  atomf_ref, efeat_ref, edist_ref, eunit_ref, nidx_ref, nz_ref):
    pid = pl.program_id(0)

    atomf_ref[...] = jnp.tanh(
        jnp.dot(f1_ref[...], w1b_ref[...], preferred_element_type=_F32))

    ia = lax.broadcasted_iota(jnp.int32, (A, A * A), 0)
    ja = lax.broadcasted_iota(jnp.int32, (A, A * A), 1)
    a1 = (ja // A == ia).astype(_F32)
    b1 = ((ja % A) == ia).astype(_F32)
    r64 = jnp.dot(lax.dot_general(a1, w1_ref[...], (((0,), (0,)), ((), ())),
                                  preferred_element_type=_F32), a1,
                  preferred_element_type=_F32)
    t64 = jnp.dot(lax.dot_general(b1, w2_ref[...], (((0,), (0,)), ((), ())),
                                  preferred_element_type=_F32), b1,
                  preferred_element_type=_F32)
    k64 = r64 * t64
    i2 = lax.broadcasted_iota(jnp.int32, (A * A, AAF), 0)
    j2 = lax.broadcasted_iota(jnp.int32, (A * A, AAF), 1)
    a2 = (j2 // F == i2).astype(_F32)
    kx = jnp.dot(lax.dot_general(a2, k64, (((0,), (0,)), ((), ())),
                                 preferred_element_type=_F32), a2,
                 preferred_element_type=_F32)
    rr = lax.broadcasted_iota(jnp.int32, (AAF, AAF), 0)
    cc = lax.broadcasted_iota(jnp.int32, (AAF, AAF), 1)
    k2 = kx * ((rr % F) == (cc % F)).astype(_F32)

    geoT = geoT_ref[...]
    zf = z_ref[...]
    slot_row = slot_row_ref[...]
    slot_col = slot_col_ref[...]

    col_i32 = lax.broadcasted_iota(jnp.int32, (1, N), 1)
    colf = col_i32.astype(_F32)
    s_colf = lax.broadcasted_iota(jnp.int32, (P, 1), 0).astype(_F32)
    s_rowf = lax.broadcasted_iota(jnp.int32, (1, P), 1).astype(_F32)
    fselT = ((lax.broadcasted_iota(jnp.int32, (F, AAF), 1) & 7)
             == lax.broadcasted_iota(jnp.int32, (F, AAF), 0)).astype(_F32)
    tri = (lax.broadcasted_iota(jnp.int32, (N, N), 0)
           <= lax.broadcasted_iota(jnp.int32, (N, N), 1)).astype(_F32)

    foldedT = []
    vm_rows = []
    for i in range(BP):
        xT = x_ref[i]
        fT = lax.dot_general(k2, xT, (((0,), (0,)), ((), ())),
                             precision=lax.Precision.DEFAULT,
                             preferred_element_type=_F32)
        foldedT.append(fT)
        n2T = jnp.dot(fselT, fT * fT,
                      precision=lax.Precision.DEFAULT,
                      preferred_element_type=_F32)
        mx = jnp.max(n2T, axis=0, keepdims=True)
        ao = -jnp.log(jnp.sqrt(mx) + 1e-6)
        vm = jnp.logical_and(ao < 12.0, col_i32 != pid * BP + i)
        vm_rows.append(vm.astype(_F32))
    m = jnp.concatenate(vm_rows, axis=0)

    vmask = m > 0.5
    cs = jnp.dot(m, tri, preferred_element_type=_F32)
    t = cs[:, N - 1:N]
    rank = jnp.where(vmask, cs - 1.0, t + colf - cs)

    for i in range(BP):
        rank_i = rank[i:i + 1, :]
        oh = (rank_i == s_colf).astype(_F32)
        t_i = t[i:i + 1, :]
        padc = jnp.where(s_colf < t_i, 1.0, 0.0) * slot_col
        padr = jnp.where(s_rowf < t_i, 1.0, 0.0) * slot_row

        efeat_ref[i] = lax.dot_general(
            oh, foldedT[i], (((1,), (1,)), ((), ())),
            precision=lax.Precision.DEFAULT,
            preferred_element_type=_F32) * padc

        g_i = jnp.sum(jnp.where(col_i32 == pid * BP + i, geoT, 0.0),
                      axis=1, keepdims=True)
        diffT = geoT - g_i
        dist = jnp.sqrt(jnp.sum(diffT * diffT, axis=0, keepdims=True)
                        + 1e-12)
        vm_i = vmask[i:i + 1, :]
        denom = jnp.where(vm_i, dist, 1.0)
        unitT = jnp.where(vm_i, diffT / denom, 0.0)

        eunit_ref[i] = lax.dot_general(
            oh, unitT, (((1,), (1,)), ((), ())),
            preferred_element_type=_F32) * padc
        edist_ref[pl.ds(i, 1), :] = lax.dot_general(
            dist, oh, (((1,), (1,)), ((), ())),
            preferred_element_type=_F32) * padr
        nidx = lax.dot_general(colf, oh, (((1,), (1,)), ((), ())),
                               preferred_element_type=_F32)
        nidx_ref[pl.ds(i, 1), :] = jnp.round(nidx).astype(jnp.int32)
        nz = lax.dot_general(zf, oh, (((1,), (1,)), ((), ())),
                             preferred_element_type=_F32) * padr
        nz_ref[pl.ds(i, 1), :] = jnp.round(nz).astype(jnp.int32)


def kernel(feat_2body, feat_1body, geometry, atomic_numbers,
           W_fold1, W_fold2, W_onebody, padding_size):
    n = feat_2body.shape[0]
    xT = jnp.transpose(feat_2body.reshape(n, n, AAF), (0, 2, 1))
    geoT = geometry.T.astype(_F32)
    zf = atomic_numbers.astype(_F32).reshape(1, n)
    slot_row = (jnp.arange(P)[None, :] < padding_size).astype(_F32)
    slot_col = slot_row.reshape(P, 1)

    grid = (n // BP,)
    outs = pl.pallas_call(
        _fused_kernel,
        grid=grid,
        in_specs=[
            pl.BlockSpec((BP, AAF, n), lambda i: (i, 0, 0)),
            pl.BlockSpec((BP, FE), lambda i: (i, 0)),
            pl.BlockSpec((3, n), lambda i: (0, 0)),
            pl.BlockSpec((1, n), lambda i: (0, 0)),
            pl.BlockSpec((A, A), lambda i: (0, 0)),
            pl.BlockSpec((A, A), lambda i: (0, 0)),
            pl.BlockSpec((FE, SP), lambda i: (0, 0)),
            pl.BlockSpec((1, P), lambda i: (0, 0)),
            pl.BlockSpec((P, 1), lambda i: (0, 0)),
        ],
        out_specs=[
            pl.BlockSpec((BP, SP), lambda i: (i, 0)),
            pl.BlockSpec((BP, P, AAF), lambda i: (i, 0, 0)),
            pl.BlockSpec((BP, P), lambda i: (i, 0)),
            pl.BlockSpec((BP, P, 3), lambda i: (i, 0, 0)),
            pl.BlockSpec((BP, P), lambda i: (i, 0)),
            pl.BlockSpec((BP, P), lambda i: (i, 0)),
        ],
        out_shape=[
            jax.ShapeDtypeStruct((n, SP), _F32),
            jax.ShapeDtypeStruct((n, P, AAF), _F32),
            jax.ShapeDtypeStruct((n, P), _F32),
            jax.ShapeDtypeStruct((n, P, 3), _F32),
            jax.ShapeDtypeStruct((n, P), jnp.int32),
            jax.ShapeDtypeStruct((n, P), jnp.int32),
        ],
        compiler_params=pltpu.CompilerParams(
            dimension_semantics=("arbitrary",)),
    )(xT, feat_1body, geoT, zf, W_fold1, W_fold2, W_onebody,
      slot_row, slot_col)

    atom_f, ef, edist, eunit, nidx, nz = outs
    return (atom_f, ef.reshape(n, P, A, A, F), edist, eunit, nidx, nz)


# speedup vs baseline: 2.3247x; 1.6116x over previous
import jax
import jax.numpy as jnp
from jax import lax
from jax.experimental import pallas as pl
from jax.experimental.pallas import tpu as pltpu

N = 256
A = 8
F = 8
FE = 16
SP = 32
P = 32
AAF = A * A * F
BP = 16

_F32 = jnp.float32


def _fused_kernel(x_ref, f1_ref, geoT_ref, z_ref, w1_ref, w2_ref, w1b_ref,
                  slot_row_ref, slot_col_ref,
                  atomf_ref, efeat_ref, edist_ref, eunit_ref, nidx_ref, nz_ref):
    pid = pl.program_id(0)

    atomf_ref[...] = jnp.tanh(
        jnp.dot(f1_ref[...], w1b_ref[...], preferred_element_type=_F32))

    ia = lax.broadcasted_iota(jnp.int32, (A, A * A), 0)
    ja = lax.broadcasted_iota(jnp.int32, (A, A * A), 1)
    a1 = (ja // A == ia).astype(_F32)
    b1 = ((ja % A) == ia).astype(_F32)
    r64 = jnp.dot(lax.dot_general(a1, w1_ref[...], (((0,), (0,)), ((), ())),
                                  preferred_element_type=_F32), a1,
                  preferred_element_type=_F32)
    t64 = jnp.dot(lax.dot_general(b1, w2_ref[...], (((0,), (0,)), ((), ())),
                                  preferred_element_type=_F32), b1,
                  preferred_element_type=_F32)
    k64 = r64 * t64
    i2 = lax.broadcasted_iota(jnp.int32, (A * A, AAF), 0)
    j2 = lax.broadcasted_iota(jnp.int32, (A * A, AAF), 1)
    a2 = (j2 // F == i2).astype(_F32)
    kx = jnp.dot(lax.dot_general(a2, k64, (((0,), (0,)), ((), ())),
                                 preferred_element_type=_F32), a2,
                 preferred_element_type=_F32)
    rr = lax.broadcasted_iota(jnp.int32, (AAF, AAF), 0)
    cc = lax.broadcasted_iota(jnp.int32, (AAF, AAF), 1)
    k2 = kx * ((rr % F) == (cc % F)).astype(_F32)

    geoT = geoT_ref[...]
    zf = z_ref[...]
    slot_row = slot_row_ref[...]
    slot_col = slot_col_ref[...]

    col_i32 = lax.broadcasted_iota(jnp.int32, (1, N), 1)
    colf = col_i32.astype(_F32)
    s_colf = lax.broadcasted_iota(jnp.int32, (P, 1), 0).astype(_F32)
    s_rowf = lax.broadcasted_iota(jnp.int32, (1, P), 1).astype(_F32)
    tri = (lax.broadcasted_iota(jnp.int32, (N, N), 0)
           <= lax.broadcasted_iota(jnp.int32, (N, N), 1)).astype(_F32)

    foldedT = []
    vm_rows = []
    for i in range(BP):
        xT = x_ref[i]
        fT = lax.dot_general(k2, xT, (((0,), (0,)), ((), ())),
                             precision=lax.Precision.DEFAULT,
                             preferred_element_type=_F32)
        foldedT.append(fT)
        sq = fT * fT
        n2T = jnp.sum(sq.reshape(A * A, F, N), axis=0)
        mx = jnp.max(n2T, axis=0, keepdims=True)
        ao = -jnp.log(jnp.sqrt(mx) + 1e-6)
        vm = jnp.logical_and(ao < 12.0, col_i32 != pid * BP + i)
        vm_rows.append(vm.astype(_F32))
    m = jnp.concatenate(vm_rows, axis=0)

    vmask = m > 0.5
    cs = jnp.dot(m, tri, preferred_element_type=_F32)
    t = cs[:, N - 1:N]
    rank = jnp.where(vmask, cs - 1.0, t + colf - cs)

    for i in range(BP):
        rank_i = rank[i:i + 1, :]
        oh = (rank_i == s_colf).astype(_F32)
        t_i = t[i:i + 1, :]
        padc = jnp.where(s_colf < t_i, 1.0, 0.0) * slot_col
        padr = jnp.where(s_rowf < t_i, 1.0, 0.0) * slot_row

        efeat_ref[i] = lax.dot_general(
            oh, foldedT[i], (((1,), (1,)), ((), ())),
            precision=lax.Precision.DEFAULT,
            preferred_element_type=_F32) * padc

        g_i = jnp.sum(jnp.where(col_i32 == pid * BP + i, geoT, 0.0),
                      axis=1, keepdims=True)
        diffT = geoT - g_i
        dist = jnp.sqrt(jnp.sum(diffT * diffT, axis=0, keepdims=True)
                        + 1e-12)
        vm_i = vmask[i:i + 1, :]
        denom = jnp.where(vm_i, dist, 1.0)
        unitT = jnp.where(vm_i, diffT / denom, 0.0)

        eunit_ref[i] = lax.dot_general(
            oh, unitT, (((1,), (1,)), ((), ())),
            preferred_element_type=_F32) * padc
        edist_ref[pl.ds(i, 1), :] = lax.dot_general(
            dist, oh, (((1,), (1,)), ((), ())),
            preferred_element_type=_F32) * padr
        nidx = lax.dot_general(colf, oh, (((1,), (1,)), ((), ())),
                               preferred_element_type=_F32)
        nidx_ref[pl.ds(i, 1), :] = jnp.round(nidx).astype(jnp.int32)
        nz = lax.dot_general(zf, oh, (((1,), (1,)), ((), ())),
                             preferred_element_type=_F32) * padr
        nz_ref[pl.ds(i, 1), :] = jnp.round(nz).astype(jnp.int32)


def kernel(feat_2body, feat_1body, geometry, atomic_numbers,
           W_fold1, W_fold2, W_onebody, padding_size):
    n = feat_2body.shape[0]
    xT = jnp.transpose(feat_2body.reshape(n, n, AAF), (0, 2, 1))
    geoT = geometry.T.astype(_F32)
    zf = atomic_numbers.astype(_F32).reshape(1, n)
    slot_row = (jnp.arange(P)[None, :] < padding_size).astype(_F32)
    slot_col = slot_row.reshape(P, 1)

    grid = (n // BP,)
    outs = pl.pallas_call(
        _fused_kernel,
        grid=grid,
        in_specs=[
            pl.BlockSpec((BP, AAF, n), lambda i: (i, 0, 0)),
            pl.BlockSpec((BP, FE), lambda i: (i, 0)),
            pl.BlockSpec((3, n), lambda i: (0, 0)),
            pl.BlockSpec((1, n), lambda i: (0, 0)),
            pl.BlockSpec((A, A), lambda i: (0, 0)),
            pl.BlockSpec((A, A), lambda i: (0, 0)),
            pl.BlockSpec((FE, SP), lambda i: (0, 0)),
            pl.BlockSpec((1, P), lambda i: (0, 0)),
            pl.BlockSpec((P, 1), lambda i: (0, 0)),
        ],
        out_specs=[
            pl.BlockSpec((BP, SP), lambda i: (i, 0)),
            pl.BlockSpec((BP, P, AAF), lambda i: (i, 0, 0)),
            pl.BlockSpec((BP, P), lambda i: (i, 0)),
            pl.BlockSpec((BP, P, 3), lambda i: (i, 0, 0)),
            pl.BlockSpec((BP, P), lambda i: (i, 0)),
            pl.BlockSpec((BP, P), lambda i: (i, 0)),
        ],
        out_shape=[
            jax.ShapeDtypeStruct((n, SP), _F32),
            jax.ShapeDtypeStruct((n, P, AAF), _F32),
            jax.ShapeDtypeStruct((n, P), _F32),
            jax.ShapeDtypeStruct((n, P, 3), _F32),
            jax.ShapeDtypeStruct((n, P), jnp.int32),
            jax.ShapeDtypeStruct((n, P), jnp.int32),
        ],
        compiler_params=pltpu.CompilerParams(
            dimension_semantics=("arbitrary",)),
    )(xT, feat_1body, geoT, zf, W_fold1, W_fold2, W_onebody,
      slot_row, slot_col)

    atom_f, ef, edist, eunit, nidx, nz = outs
    return (atom_f, ef.reshape(n, P, A, A, F), edist, eunit, nidx, nz)


# speedup vs baseline: 2.3756x; 1.0219x over previous
import jax
import jax.numpy as jnp
from jax import lax
from jax.experimental import pallas as pl
from jax.experimental.pallas import tpu as pltpu

N = 256
A = 8
F = 8
FE = 16
SP = 32
P = 32
AAF = A * A * F
BP = 16

_F32 = jnp.float32


def _fused_kernel(x_ref, f1_ref, geoT_ref, z_ref, w1_ref, w2_ref, w1b_ref,
                  slot_row_ref, slot_col_ref,
                  atomf_ref, efeat_ref, edist_ref, eunit_ref, nidx_ref, nz_ref):
    pid = pl.program_id(0)

    atomf_ref[...] = jnp.tanh(
        jnp.dot(f1_ref[...], w1b_ref[...], preferred_element_type=_F32))

    ia = lax.broadcasted_iota(jnp.int32, (A, A * A), 0)
    ja = lax.broadcasted_iota(jnp.int32, (A, A * A), 1)
    a1 = (ja // A == ia).astype(_F32)
    b1 = ((ja % A) == ia).astype(_F32)
    r64 = jnp.dot(lax.dot_general(a1, w1_ref[...], (((0,), (1,)), ((), ())),
                                  preferred_element_type=_F32), a1,
                  preferred_element_type=_F32)
    t64 = jnp.dot(lax.dot_general(b1, w2_ref[...], (((0,), (1,)), ((), ())),
                                  preferred_element_type=_F32), b1,
                  preferred_element_type=_F32)
    k64 = r64 * t64
    i2 = lax.broadcasted_iota(jnp.int32, (A * A, AAF), 0)
    j2 = lax.broadcasted_iota(jnp.int32, (A * A, AAF), 1)
    a2 = (j2 // F == i2).astype(_F32)
    kx = jnp.dot(lax.dot_general(a2, k64, (((0,), (0,)), ((), ())),
                                 preferred_element_type=_F32), a2,
                 preferred_element_type=_F32)
    rr = lax.broadcasted_iota(jnp.int32, (AAF, AAF), 0)
    cc = lax.broadcasted_iota(jnp.int32, (AAF, AAF), 1)
    k2 = kx * ((rr % F) == (cc % F)).astype(_F32)

    geoT = geoT_ref[...]
    zf = z_ref[...]
    slot_row = slot_row_ref[...]
    slot_col = slot_col_ref[...]

    col_i32 = lax.broadcasted_iota(jnp.int32, (1, N), 1)
    colf = col_i32.astype(_F32)
    s_colf = lax.broadcasted_iota(jnp.int32, (P, 1), 0).astype(_F32)
    s_rowf = lax.broadcasted_iota(jnp.int32, (1, P), 1).astype(_F32)
    tri = (lax.broadcasted_iota(jnp.int32, (N, N), 0)
           <= lax.broadcasted_iota(jnp.int32, (N, N), 1)).astype(_F32)

    foldedT = []
    vm_rows = []
    for i in range(BP):
        xT = x_ref[i]
        fT = jnp.dot(k2, xT, precision=lax.Precision.DEFAULT,
                     preferred_element_type=_F32)
        foldedT.append(fT)
        sq = fT * fT
        n2T = jnp.sum(sq.reshape(A * A, F, N), axis=0)
        mx = jnp.max(n2T, axis=0, keepdims=True)
        ao = -jnp.log(jnp.sqrt(mx) + 1e-6)
        vm = jnp.logical_and(ao < 12.0, col_i32 != pid * BP + i)
        vm_rows.append(vm.astype(_F32))
    m = jnp.concatenate(vm_rows, axis=0)

    vmask = m > 0.5
    cs = jnp.dot(m, tri, preferred_element_type=_F32)
    t = cs[:, N - 1:N]
    rank = jnp.where(vmask, cs - 1.0, t + colf - cs)

    for i in range(BP):
        rank_i = rank[i:i + 1, :]
        oh = (rank_i == s_colf).astype(_F32)
        t_i = t[i:i + 1, :]
        padc = jnp.where(s_colf < t_i, 1.0, 0.0) * slot_col
        padr = jnp.where(s_rowf < t_i, 1.0, 0.0) * slot_row

        efeat_ref[i] = lax.dot_general(
            oh, foldedT[i], (((1,), (1,)), ((), ())),
            precision=lax.Precision.DEFAULT,
            preferred_element_type=_F32) * padc

        g_i = jnp.sum(jnp.where(col_i32 == pid * BP + i, geoT, 0.0),
                      axis=1, keepdims=True)
        diffT = geoT - g_i
        dist = jnp.sqrt(jnp.sum(diffT * diffT, axis=0, keepdims=True)
                        + 1e-12)
        vm_i = vmask[i:i + 1, :]
        denom = jnp.where(vm_i, dist, 1.0)
        unitT = jnp.where(vm_i, diffT / denom, 0.0)

        eunit_ref[i] = lax.dot_general(
            oh, unitT, (((1,), (1,)), ((), ())),
            preferred_element_type=_F32) * padc
        edist_ref[pl.ds(i, 1), :] = lax.dot_general(
            dist, oh, (((1,), (1,)), ((), ())),
            preferred_element_type=_F32) * padr
        nidx = lax.dot_general(colf, oh, (((1,), (1,)), ((), ())),
                               preferred_element_type=_F32)
        nidx_ref[pl.ds(i, 1), :] = jnp.round(nidx).astype(jnp.int32)
        nz = lax.dot_general(zf, oh, (((1,), (1,)), ((), ())),
                             preferred_element_type=_F32) * padr
        nz_ref[pl.ds(i, 1), :] = jnp.round(nz).astype(jnp.int32)


def kernel(feat_2body, feat_1body, geometry, atomic_numbers,
           W_fold1, W_fold2, W_onebody, padding_size):
    n = feat_2body.shape[0]
    xT = jnp.transpose(feat_2body.reshape(n, n, AAF), (0, 2, 1))
    geoT = geometry.T.astype(_F32)
    zf = atomic_numbers.astype(_F32).reshape(1, n)
    slot_row = (jnp.arange(P)[None, :] < padding_size).astype(_F32)
    slot_col = slot_row.reshape(P, 1)

    grid = (n // BP,)
    outs = pl.pallas_call(
        _fused_kernel,
        grid=grid,
        in_specs=[
            pl.BlockSpec((BP, AAF, n), lambda i: (i, 0, 0)),
            pl.BlockSpec((BP, FE), lambda i: (i, 0)),
            pl.BlockSpec((3, n), lambda i: (0, 0)),
            pl.BlockSpec((1, n), lambda i: (0, 0)),
            pl.BlockSpec((A, A), lambda i: (0, 0)),
            pl.BlockSpec((A, A), lambda i: (0, 0)),
            pl.BlockSpec((FE, SP), lambda i: (0, 0)),
            pl.BlockSpec((1, P), lambda i: (0, 0)),
            pl.BlockSpec((P, 1), lambda i: (0, 0)),
        ],
        out_specs=[
            pl.BlockSpec((BP, SP), lambda i: (i, 0)),
            pl.BlockSpec((BP, P, AAF), lambda i: (i, 0, 0)),
            pl.BlockSpec((BP, P), lambda i: (i, 0)),
            pl.BlockSpec((BP, P, 3), lambda i: (i, 0, 0)),
            pl.BlockSpec((BP, P), lambda i: (i, 0)),
            pl.BlockSpec((BP, P), lambda i: (i, 0)),
        ],
        out_shape=[
            jax.ShapeDtypeStruct((n, SP), _F32),
            jax.ShapeDtypeStruct((n, P, AAF), _F32),
            jax.ShapeDtypeStruct((n, P), _F32),
            jax.ShapeDtypeStruct((n, P, 3), _F32),
            jax.ShapeDtypeStruct((n, P), jnp.int32),
            jax.ShapeDtypeStruct((n, P), jnp.int32),
        ],
        compiler_params=pltpu.CompilerParams(
            dimension_semantics=("arbitrary",)),
    )(xT, feat_1body, geoT, zf, W_fold1, W_fold2, W_onebody,
      slot_row, slot_col)

    atom_f, ef, edist, eunit, nidx, nz = outs
    return (atom_f, ef.reshape(n, P, A, A, F), edist, eunit, nidx, nz)
